# baseline (speedup 1.0000x reference)
import functools

import jax
import jax.numpy as jnp
from jax import lax
from jax.experimental import pallas as pl
from jax.experimental.pallas import tpu as pltpu
from jax.experimental.pallas import tpu_sc as plsc

N = 10000
E = 160000
H = 256
HH = 128
WPE = HH // 2
NB = 10
NC = 10
R = 1000
GRID = N // R

NTILES = 16
NCORES = 2
CHU = 16
WIN = 2048
UPW = WIN // CHU
NW = 10
EPT = NW * WIN
E2 = 2 * E
E2P = NTILES * EPT
ACC_ROWS = 10240
PAD_ROW = N + 64
ZROWS = 16


def _gs_body(y1p_hbm, src2_hbm, dst_hbm, out_hbm, acc,
             swin, dwin, pk0, pk1, rows0, rows1, zbuf,
             sem0, sem1, ssem0, ssem1):
    c = lax.axis_index("c")
    s = lax.axis_index("s")

    pltpu.sync_copy(src2_hbm.at[(c * NTILES + s) * NW], swin)
    pltpu.sync_copy(dst_hbm.at[s * NW], dwin)
    pltpu.async_copy(y1p_hbm.at[swin.at[pl.ds(0, CHU)]], pk0, sem0)
    pltpu.async_copy(y1p_hbm.at[swin.at[pl.ds(CHU, CHU)]], pk1, sem1)

    def _zb(i, _):
        r = i // (HH // 16)
        l = i % (HH // 16)
        zbuf[r, pl.ds(l * 16, 16)] = jnp.zeros((16,), jnp.float32)
        return _
    lax.fori_loop(0, ZROWS * (HH // 16), _zb, 0)

    rows_per_tile = ACC_ROWS // NTILES

    def _zero(i, _):
        pltpu.sync_copy(zbuf, acc.at[pl.ds(s * rows_per_tile + i * ZROWS, ZROWS)])
        return _
    lax.fori_loop(0, rows_per_tile // ZROWS, _zero, 0)

    plsc.subcore_barrier()

    def _unit(w, u, pk, rows, gsem, ssem):
        pltpu.make_async_copy(y1p_hbm.at[swin.at[pl.ds(0, CHU)]], pk,
                              gsem).wait()

        @pl.when(w * UPW + u >= 2)
        def _():
            pltpu.make_async_copy(rows, acc.at[dwin.at[0]], ssem).wait()

        for e in range(CHU):
            for k in range(WPE // 16):
                iv = pk[e, pl.ds(k * 16, 16)]
                rows[e, pl.ds(k * 16, 16)] = lax.bitcast_convert_type(
                    iv << 16, jnp.float32)
                rows[e, pl.ds(WPE + k * 16, 16)] = lax.bitcast_convert_type(
                    lax.shift_right_logical(iv, 16) << 16, jnp.float32)

        pltpu.async_copy(rows, acc.at[dwin.at[u]], ssem, add=True)

        @pl.when(u + 2 < UPW)
        def _():
            pltpu.async_copy(y1p_hbm.at[swin.at[pl.ds((u + 2) * CHU, CHU)]],
                             pk, gsem)

    for w in range(NW):
        if w > 0:
            pltpu.sync_copy(src2_hbm.at[(c * NTILES + s) * NW + w], swin)
            pltpu.sync_copy(dst_hbm.at[s * NW + w], dwin)
            pltpu.async_copy(y1p_hbm.at[swin.at[pl.ds(0, CHU)]], pk0, sem0)
            pltpu.async_copy(y1p_hbm.at[swin.at[pl.ds(CHU, CHU)]], pk1, sem1)

        def _pair(p, carry):
            _unit(w, 2 * p, pk0, rows0, sem0, ssem0)
            _unit(w, 2 * p + 1, pk1, rows1, sem1, ssem1)
            return carry
        lax.fori_loop(0, UPW // 2, _pair, 0)

    pltpu.make_async_copy(rows0, acc.at[dwin.at[0]], ssem0).wait()
    pltpu.make_async_copy(rows1, acc.at[dwin.at[1]], ssem1).wait()

    plsc.subcore_barrier()

    @pl.when(s < NTILES - 1)
    def _():
        pltpu.sync_copy(acc.at[pl.ds(s * 640, 640)],
                        out_hbm.at[pl.ds(c * N + s * 640, 640)])

    @pl.when(s == NTILES - 1)
    def _():
        pltpu.sync_copy(acc.at[pl.ds((NTILES - 1) * 640, 400)],
                        out_hbm.at[pl.ds(c * N + (NTILES - 1) * 640, 400)])


_gs_call = functools.partial(
    pl.kernel,
    out_type=jax.ShapeDtypeStruct((NCORES * N, HH), jnp.float32),
    mesh=plsc.VectorSubcoreMesh(core_axis_name="c", subcore_axis_name="s"),
    compiler_params=pltpu.CompilerParams(use_tc_tiling_on_sc=False),
    scratch_types=[
        pltpu.VMEM_SHARED((ACC_ROWS, HH), jnp.float32),
        pltpu.VMEM((WIN,), jnp.int32),
        pltpu.VMEM((UPW, CHU), jnp.int32),
        pltpu.VMEM((CHU, WPE), jnp.int32),
        pltpu.VMEM((CHU, WPE), jnp.int32),
        pltpu.VMEM((CHU, HH), jnp.float32),
        pltpu.VMEM((CHU, HH), jnp.float32),
        pltpu.VMEM((ZROWS, HH), jnp.float32),
        pltpu.SemaphoreType.DMA,
        pltpu.SemaphoreType.DMA,
        pltpu.SemaphoreType.DMA,
        pltpu.SemaphoreType.DMA,
    ],
)(_gs_body)


def _pack_y1(y):
    r = lax.bitcast_convert_type(y[:, H:], jnp.int32)
    b = (r + 0x7FFF + ((r >> 16) & 1)) >> 16
    lo = b & 0xFFFF
    return [lo[:, c * HH:c * HH + WPE]
            | (b[:, c * HH + WPE:(c + 1) * HH] << 16) for c in (0, 1)]


def _layer1_body(x_ref, w_ref, b_ref, y0_ref, y1p_ref):
    y = jnp.dot(x_ref[...], w_ref[...], preferred_element_type=jnp.float32)
    y = y + b_ref[...]
    y0_ref[...] = y[:, :H]
    y1p = _pack_y1(y)
    y1p_ref[0] = y1p[0]
    y1p_ref[1] = y1p[1]


def _layerB_body(y0p_ref, gsp_ref, w_ref, b_ref, y0_ref, y1p_ref):
    gs = jnp.concatenate([gsp_ref[0], gsp_ref[1]], axis=1)
    x = jnp.maximum(y0p_ref[...] + gs, 0.0)
    y = jnp.dot(x, w_ref[...], preferred_element_type=jnp.float32)
    y = y + b_ref[...]
    y0_ref[...] = y[:, :H]
    y1p = _pack_y1(y)
    y1p_ref[0] = y1p[0]
    y1p_ref[1] = y1p[1]


def _layer1(x, w, b):
    return pl.pallas_call(
        _layer1_body,
        grid=(GRID,),
        in_specs=[
            pl.BlockSpec((R, x.shape[1]), lambda i: (i, 0)),
            pl.BlockSpec((x.shape[1], 2 * H), lambda i: (0, 0)),
            pl.BlockSpec((1, 2 * H), lambda i: (0, 0)),
        ],
        out_specs=[
            pl.BlockSpec((R, H), lambda i: (i, 0)),
            pl.BlockSpec((2, R, WPE), lambda i: (0, i, 0)),
        ],
        out_shape=[
            jax.ShapeDtypeStruct((N, H), jnp.float32),
            jax.ShapeDtypeStruct((2, N, WPE), jnp.int32),
        ],
    )(x, w, b)


def _layerB(y0p, gsp, w, b):
    return pl.pallas_call(
        _layerB_body,
        grid=(GRID,),
        in_specs=[
            pl.BlockSpec((R, H), lambda i: (i, 0)),
            pl.BlockSpec((2, R, HH), lambda i: (0, i, 0)),
            pl.BlockSpec((H, 2 * H), lambda i: (0, 0)),
            pl.BlockSpec((1, 2 * H), lambda i: (0, 0)),
        ],
        out_specs=[
            pl.BlockSpec((R, H), lambda i: (i, 0)),
            pl.BlockSpec((2, R, WPE), lambda i: (0, i, 0)),
        ],
        out_shape=[
            jax.ShapeDtypeStruct((N, H), jnp.float32),
            jax.ShapeDtypeStruct((2, N, WPE), jnp.int32),
        ],
    )(y0p, gsp, w, b)


def _pool_body(y0p_ref, gsp_ref, seg_ref, fw1_ref, fb1_ref, fw2_ref, fb2_ref,
               out_ref, acc, cnt):
    i = pl.program_id(0)

    @pl.when(i == 0)
    def _():
        acc[...] = jnp.zeros_like(acc)
        cnt[...] = jnp.zeros_like(cnt)

    gs = jnp.concatenate([gsp_ref[0], gsp_ref[1]], axis=1)
    x = jnp.maximum(y0p_ref[...] + gs, 0.0)
    seg = seg_ref[...].reshape(1, R)
    sel = jnp.broadcast_to(seg, (16, R)) == lax.broadcasted_iota(
        jnp.int32, (16, R), 0)
    onehot = sel.astype(jnp.float32)
    acc[...] += jnp.dot(onehot, x, preferred_element_type=jnp.float32)
    csum = jnp.sum(onehot, axis=1, keepdims=True)
    cnt[...] += jnp.broadcast_to(csum, cnt.shape)

    @pl.when(i == GRID - 1)
    def _():
        counts = jnp.maximum(cnt[:, :1], 1.0)
        mesh_feats = acc[...] / counts
        h = jnp.dot(mesh_feats, fw1_ref[...], preferred_element_type=jnp.float32)
        h = jnp.maximum(h + fb1_ref[...], 0.0)
        o = jnp.dot(h, fw2_ref[...], preferred_element_type=jnp.float32)
        o = o + fb2_ref[...]
        out_ref[...] = o[:NB, :]


def _pool(y0p, gsp, seg3, fw1, fb1, fw2, fb2):
    return pl.pallas_call(
        _pool_body,
        grid=(GRID,),
        in_specs=[
            pl.BlockSpec((R, H), lambda i: (i, 0)),
            pl.BlockSpec((2, R, HH), lambda i: (0, i, 0)),
            pl.BlockSpec((1, 1, R), lambda i: (i, 0, 0)),
            pl.BlockSpec((H, H), lambda i: (0, 0)),
            pl.BlockSpec((1, H), lambda i: (0, 0)),
            pl.BlockSpec((H, NC), lambda i: (0, 0)),
            pl.BlockSpec((1, NC), lambda i: (0, 0)),
        ],
        out_specs=pl.BlockSpec((NB, NC), lambda i: (0, 0)),
        out_shape=jax.ShapeDtypeStruct((NB, NC), jnp.float32),
        scratch_shapes=[
            pltpu.VMEM((16, H), jnp.float32),
            pltpu.VMEM((16, 128), jnp.float32),
        ],
    )(y0p, gsp, seg3, fw1, fb1, fw2, fb2)


def kernel(verts, edges, segment_ids,
           w0_1, b0_1, w1_1, b1_1,
           w0_2, b0_2, w1_2, b1_2,
           w0_3, b0_3, w1_3, b1_3,
           fc1_w, fc1_b, fc2_w, fc2_b):
    src = jnp.concatenate([edges[:, 1], edges[:, 0]])
    dst = jnp.concatenate([edges[:, 0], edges[:, 1]])
    pad = E2P - E2
    srcp = jnp.concatenate([src, jnp.zeros((pad,), jnp.int32)])
    dstp = jnp.concatenate([dst, jnp.full((pad,), PAD_ROW, jnp.int32)])
    src2 = jnp.concatenate([srcp, srcp + N]).reshape(
        NCORES * NTILES * NW, WIN)
    dstp = dstp.reshape(NTILES * NW, UPW, CHU)

    def _wb(w0, b0, w1, b1):
        wc = jnp.concatenate([w0.T, w1.T], axis=1)
        bc = jnp.concatenate([b0, b1]).reshape(1, 2 * H)
        return wc, bc

    w1c, b1c = _wb(w0_1, b0_1, w1_1, b1_1)
    w2c, b2c = _wb(w0_2, b0_2, w1_2, b1_2)
    w3c, b3c = _wb(w0_3, b0_3, w1_3, b1_3)
    seg3 = segment_ids.reshape(GRID, 1, R)

    y0, y1p = _layer1(verts, w1c, b1c)
    gs = _gs_call(y1p.reshape(2 * N, WPE), src2, dstp)
    y0, y1p = _layerB(y0, gs.reshape(2, N, HH), w2c, b2c)
    gs = _gs_call(y1p.reshape(2 * N, WPE), src2, dstp)
    y0, y1p = _layerB(y0, gs.reshape(2, N, HH), w3c, b3c)
    gs = _gs_call(y1p.reshape(2 * N, WPE), src2, dstp)

    return _pool(y0, gs.reshape(2, N, HH), seg3,
                 fc1_w.T, fc1_b.reshape(1, H),
                 fc2_w.T, fc2_b.reshape(1, NC))
